# baseline (speedup 1.0000x reference)
import functools

import jax
import jax.numpy as jnp
from jax.experimental import pallas as pl
from jax.experimental.pallas import tpu as pltpu

_TV = 4096


def _fused_kernel(idx_ref, table_ref, w1_ref, b1_ref, w2t_ref, b2_ref,
                  out_ref, emb_ref, h_ref, sems, *, ctx, embed, vocab, nv):
    j = pl.program_id(0)

    @pl.when(j == 0)
    def _compute_hidden():
        copies = []
        for c in range(ctx):
            cp = pltpu.make_async_copy(
                table_ref.at[pl.ds(idx_ref[c], 1), :],
                emb_ref.at[:, pl.ds(c * embed, embed)],
                sems.at[c],
            )
            cp.start()
            copies.append(cp)
        for cp in copies:
            cp.wait()
        acc = jax.lax.dot_general(
            emb_ref[...], w1_ref[...],
            dimension_numbers=(((1,), (0,)), ((), ())),
            preferred_element_type=jnp.float32,
        )
        h_ref[...] = jnp.maximum(acc + b1_ref[...], 0.0)

    logits = jax.lax.dot_general(
        h_ref[...], w2t_ref[...],
        dimension_numbers=(((1,), (1,)), ((), ())),
        preferred_element_type=jnp.float32,
    ) + b2_ref[...]

    rem = vocab - (nv - 1) * _TV

    @pl.when(j < nv - 1)
    def _store_full():
        out_ref[:, pl.ds(pl.multiple_of(j * _TV, _TV), _TV)] = logits

    @pl.when(j == nv - 1)
    def _store_tail_and_normalize():
        out_ref[:, pl.ds((nv - 1) * _TV, rem)] = logits[:, :rem]
        x = out_ref[...]
        m = jnp.max(x)
        lse = m + jnp.log(jnp.sum(jnp.exp(x - m)))
        out_ref[...] = x - lse


def kernel(inputs, table, W1, b1, W2, b2):
    vocab, embed = table.shape
    ctx = inputs.shape[0]
    hidden = W1.shape[1]
    nv = pl.cdiv(vocab, _TV)

    idx = inputs.astype(jnp.int32)
    b1r = b1.reshape(1, hidden)
    b2r = b2.reshape(1, vocab)
    W2T = W2.T

    grid_spec = pltpu.PrefetchScalarGridSpec(
        num_scalar_prefetch=1,
        grid=(nv,),
        in_specs=[
            pl.BlockSpec(memory_space=pl.ANY),
            pl.BlockSpec((ctx * embed, hidden), lambda j, idx: (0, 0)),
            pl.BlockSpec((1, hidden), lambda j, idx: (0, 0)),
            pl.BlockSpec((_TV, hidden), lambda j, idx: (j, 0)),
            pl.BlockSpec((1, _TV), lambda j, idx: (0, j)),
        ],
        out_specs=pl.BlockSpec((1, vocab), lambda j, idx: (0, 0)),
        scratch_shapes=[
            pltpu.VMEM((1, ctx * embed), jnp.float32),
            pltpu.VMEM((1, hidden), jnp.float32),
            pltpu.SemaphoreType.DMA((ctx,)),
        ],
    )

    return pl.pallas_call(
        functools.partial(_fused_kernel, ctx=ctx, embed=embed,
                          vocab=vocab, nv=nv),
        grid_spec=grid_spec,
        out_shape=jax.ShapeDtypeStruct((1, vocab), jnp.float32),
        compiler_params=pltpu.CompilerParams(
            dimension_semantics=("arbitrary",),
            vmem_limit_bytes=64 * 1024 * 1024,
        ),
    )(idx, table, W1, b1r, W2T, b2r)


# speedup vs baseline: 1.1099x; 1.1099x over previous
import functools

import jax
import jax.numpy as jnp
from jax.experimental import pallas as pl
from jax.experimental.pallas import tpu as pltpu

_TV = 8192


def _fused_kernel(idx_ref, table_ref, w1_ref, b1_ref, w2t_ref, b2_ref,
                  out_ref, emb_ref, h_ref, sems, *, ctx, embed, vocab, nv):
    j = pl.program_id(0)

    @pl.when(j == 0)
    def _compute_hidden():
        copies = []
        for c in range(ctx):
            cp = pltpu.make_async_copy(
                table_ref.at[pl.ds(idx_ref[c], 1), :],
                emb_ref.at[:, pl.ds(c * embed, embed)],
                sems.at[c],
            )
            cp.start()
            copies.append(cp)
        for cp in copies:
            cp.wait()
        acc = jax.lax.dot_general(
            emb_ref[...], w1_ref[...],
            dimension_numbers=(((1,), (0,)), ((), ())),
            preferred_element_type=jnp.float32,
        )
        h_ref[...] = jnp.maximum(acc + b1_ref[...], 0.0)

    logits = jax.lax.dot_general(
        h_ref[...], w2t_ref[...],
        dimension_numbers=(((1,), (1,)), ((), ())),
        preferred_element_type=jnp.float32,
    ) + b2_ref[...]

    rem = vocab - (nv - 1) * _TV

    @pl.when(j < nv - 1)
    def _store_full():
        out_ref[:, pl.ds(pl.multiple_of(j * _TV, _TV), _TV)] = logits

    @pl.when(j == nv - 1)
    def _store_tail_and_normalize():
        out_ref[:, pl.ds((nv - 1) * _TV, rem)] = logits[:, :rem]
        x = out_ref[...]
        m = jnp.max(x)
        lse = m + jnp.log(jnp.sum(jnp.exp(x - m)))
        out_ref[...] = x - lse


def kernel(inputs, table, W1, b1, W2, b2):
    vocab, embed = table.shape
    ctx = inputs.shape[0]
    hidden = W1.shape[1]
    nv = pl.cdiv(vocab, _TV)

    idx = inputs.astype(jnp.int32)
    b1r = b1.reshape(1, hidden)
    b2r = b2.reshape(1, vocab)
    W2T = W2.T

    grid_spec = pltpu.PrefetchScalarGridSpec(
        num_scalar_prefetch=1,
        grid=(nv,),
        in_specs=[
            pl.BlockSpec(memory_space=pl.ANY),
            pl.BlockSpec((ctx * embed, hidden), lambda j, idx: (0, 0)),
            pl.BlockSpec((1, hidden), lambda j, idx: (0, 0)),
            pl.BlockSpec((_TV, hidden), lambda j, idx: (j, 0)),
            pl.BlockSpec((1, _TV), lambda j, idx: (0, j)),
        ],
        out_specs=pl.BlockSpec((1, vocab), lambda j, idx: (0, 0)),
        scratch_shapes=[
            pltpu.VMEM((1, ctx * embed), jnp.float32),
            pltpu.VMEM((1, hidden), jnp.float32),
            pltpu.SemaphoreType.DMA((ctx,)),
        ],
    )

    return pl.pallas_call(
        functools.partial(_fused_kernel, ctx=ctx, embed=embed,
                          vocab=vocab, nv=nv),
        grid_spec=grid_spec,
        out_shape=jax.ShapeDtypeStruct((1, vocab), jnp.float32),
        compiler_params=pltpu.CompilerParams(
            dimension_semantics=("arbitrary",),
            vmem_limit_bytes=64 * 1024 * 1024,
        ),
    )(idx, table, W1, b1r, W2T, b2r)


# speedup vs baseline: 1.1353x; 1.0228x over previous
import functools

import jax
import jax.numpy as jnp
from jax.experimental import pallas as pl
from jax.experimental.pallas import tpu as pltpu

_TV = 8192


def _fused_kernel(idx_ref, table_ref, w1_ref, b1_ref, w2t_ref, b2_ref,
                  out_ref, emb_ref, h_ref, m_ref, s_ref, sems,
                  *, ctx, embed, vocab, nv):
    j = pl.program_id(0)

    @pl.when(j == 0)
    def _compute_hidden():
        copies = []
        for c in range(ctx):
            cp = pltpu.make_async_copy(
                table_ref.at[pl.ds(idx_ref[c], 1), :],
                emb_ref.at[:, pl.ds(c * embed, embed)],
                sems.at[c],
            )
            cp.start()
            copies.append(cp)
        for cp in copies:
            cp.wait()
        acc = jax.lax.dot_general(
            emb_ref[...], w1_ref[...],
            dimension_numbers=(((1,), (0,)), ((), ())),
            preferred_element_type=jnp.float32,
        )
        h_ref[...] = jnp.maximum(acc + b1_ref[...], 0.0)
        m_ref[...] = jnp.full((1, _TV), -jnp.inf, jnp.float32)
        s_ref[...] = jnp.zeros((1, _TV), jnp.float32)

    logits = jax.lax.dot_general(
        h_ref[...], w2t_ref[...],
        dimension_numbers=(((1,), (1,)), ((), ())),
        preferred_element_type=jnp.float32,
    ) + b2_ref[...]

    rem = vocab - (nv - 1) * _TV

    lane = jax.lax.broadcasted_iota(jnp.int32, (1, _TV), 1)
    valid = jnp.where(lane < rem, logits, -jnp.inf)
    tile = jnp.where(j == nv - 1, valid, logits)
    m_old = m_ref[...]
    m_new = jnp.maximum(m_old, tile)
    scale = jnp.where(jnp.isneginf(m_old), 0.0, jnp.exp(m_old - m_new))
    s_ref[...] = s_ref[...] * scale + jnp.exp(tile - m_new)
    m_ref[...] = m_new

    @pl.when(j < nv - 1)
    def _store_full():
        out_ref[:, pl.ds(pl.multiple_of(j * _TV, _TV), _TV)] = logits

    @pl.when(j == nv - 1)
    def _store_tail_and_normalize():
        out_ref[:, pl.ds((nv - 1) * _TV, rem)] = logits[:, :rem]
        m_vec = m_ref[...]
        mg = jnp.max(m_vec)
        s_tot = jnp.sum(s_ref[...] * jnp.exp(m_vec - mg))
        lse = mg + jnp.log(s_tot)
        out_ref[...] = out_ref[...] - lse


def kernel(inputs, table, W1, b1, W2, b2):
    vocab, embed = table.shape
    ctx = inputs.shape[0]
    hidden = W1.shape[1]
    nv = pl.cdiv(vocab, _TV)

    idx = inputs.astype(jnp.int32)
    b1r = b1.reshape(1, hidden)
    b2r = b2.reshape(1, vocab)
    W2T = W2.T

    grid_spec = pltpu.PrefetchScalarGridSpec(
        num_scalar_prefetch=1,
        grid=(nv,),
        in_specs=[
            pl.BlockSpec(memory_space=pl.ANY),
            pl.BlockSpec((ctx * embed, hidden), lambda j, idx: (0, 0)),
            pl.BlockSpec((1, hidden), lambda j, idx: (0, 0)),
            pl.BlockSpec((_TV, hidden), lambda j, idx: (j, 0)),
            pl.BlockSpec((1, _TV), lambda j, idx: (0, j)),
        ],
        out_specs=pl.BlockSpec((1, vocab), lambda j, idx: (0, 0)),
        scratch_shapes=[
            pltpu.VMEM((1, ctx * embed), jnp.float32),
            pltpu.VMEM((1, hidden), jnp.float32),
            pltpu.VMEM((1, _TV), jnp.float32),
            pltpu.VMEM((1, _TV), jnp.float32),
            pltpu.SemaphoreType.DMA((ctx,)),
        ],
    )

    return pl.pallas_call(
        functools.partial(_fused_kernel, ctx=ctx, embed=embed,
                          vocab=vocab, nv=nv),
        grid_spec=grid_spec,
        out_shape=jax.ShapeDtypeStruct((1, vocab), jnp.float32),
        compiler_params=pltpu.CompilerParams(
            dimension_semantics=("arbitrary",),
            vmem_limit_bytes=64 * 1024 * 1024,
        ),
    )(idx, table, W1, b1r, W2T, b2r)
